# Initial kernel scaffold; baseline (speedup 1.0000x reference)
import jax
import jax.numpy as jnp
from jax.experimental import pallas as pl


def kernel(x, W_patch, b_patch, W_cls, b_cls, W1, b1, W2, b2, W_fc, b_fc):
    raise NotImplementedError("write your pallas kernel here")


# speedup vs baseline: 15.0776x; 15.0776x over previous
import jax
import jax.numpy as jnp
from jax import lax
from jax.experimental import pallas as pl
from jax.experimental.pallas import tpu as pltpu

_P = 16
_TOP_K = 2
_KNN = 8


def _graph_kernel(patches_ref, Wp_ref, bp_ref, Wcls_ref, bcls_ref,
                  topk_ref, A_ref, agg_ref):
    f = jnp.dot(patches_ref[0], Wp_ref[...],
                preferred_element_type=jnp.float32) + bp_ref[...]
    n, feat = f.shape

    fm = jnp.mean(f, axis=0, keepdims=True)
    lg = jnp.dot(fm, Wcls_ref[...],
                 preferred_element_type=jnp.float32) + bcls_ref[...]
    c = lg.shape[1]
    iota_c = lax.broadcasted_iota(jnp.int32, (1, c), 1)
    m1 = jnp.max(lg, axis=1, keepdims=True)
    i1 = jnp.min(jnp.where(lg == m1, iota_c, c), axis=1, keepdims=True)
    lg2 = jnp.where(iota_c == i1, -jnp.inf, lg)
    m2 = jnp.max(lg2, axis=1, keepdims=True)
    i2 = jnp.min(jnp.where(lg2 == m2, iota_c, c), axis=1, keepdims=True)
    iota_2 = lax.broadcasted_iota(jnp.int32, (1, 1, _TOP_K), 2)
    topk_ref[...] = jnp.where(iota_2 == 0, i1.reshape(1, 1, 1),
                              i2.reshape(1, 1, 1))

    gram = lax.dot_general(f, f, (((1,), (1,)), ((), ())),
                           preferred_element_type=jnp.float32)
    rows = lax.broadcasted_iota(jnp.int32, (n, n), 0)
    cols = lax.broadcasted_iota(jnp.int32, (n, n), 1)
    eye = rows == cols
    sq_col = jnp.sum(jnp.where(eye, gram, 0.0), axis=0, keepdims=True)
    d = sq_col - 2.0 * gram
    d = jnp.where(eye, d + 1e9, d)

    work = d
    vals = []
    nbrs = []
    for _ in range(_KNN):
        m = jnp.min(work, axis=1, keepdims=True)
        am = jnp.min(jnp.where(work == m, cols, n), axis=1, keepdims=True)
        vals.append(m)
        nbrs.append(am)
        work = jnp.where(cols == am, jnp.inf, work)
    v = jnp.concatenate(vals, axis=1)

    z = -v * (1.0 / jnp.sqrt(jnp.float32(feat)))
    z = z - jnp.max(z, axis=1, keepdims=True)
    e = jnp.exp(z)
    w = e / jnp.sum(e, axis=1, keepdims=True)

    a = jnp.zeros((n, n), jnp.float32)
    for k in range(_KNN):
        a = a + jnp.where(cols == nbrs[k], w[:, k:k + 1], 0.0)
    A_ref[0] = a
    agg_ref[0] = jnp.dot(a, f, preferred_element_type=jnp.float32)


def _expert_kernel(idx_ref, A_ref, agg_ref, W1_ref, b1_ref, W2_ref, b2_ref,
                   Wfc_ref, bfc_ref, out_ref):
    k = pl.program_id(0) % _TOP_K
    a = A_ref[0]
    h = jnp.maximum(
        jnp.dot(agg_ref[0], W1_ref[0],
                preferred_element_type=jnp.float32) + b1_ref[0], 0.0)
    agg2 = jnp.dot(a, h, preferred_element_type=jnp.float32)
    o = jnp.dot(agg2, W2_ref[0],
                preferred_element_type=jnp.float32) + b2_ref[0]
    om = jnp.mean(o, axis=0, keepdims=True)
    y = (1.0 / _TOP_K) * (
        jnp.dot(om, Wfc_ref[...], preferred_element_type=jnp.float32)
        + bfc_ref[...])
    y = y.reshape(1, 1, -1)

    @pl.when(k == 0)
    def _():
        out_ref[...] = y

    @pl.when(k != 0)
    def _():
        out_ref[...] = out_ref[...] + y


def kernel(x, W_patch, b_patch, W_cls, b_cls, W1, b1, W2, b2, W_fc, b_fc):
    bn, c_in, hw, _ = x.shape
    p = _P
    g = hw // p
    n = g * g
    pd = c_in * p * p
    feat = W_patch.shape[1]
    c = W_cls.shape[1]
    hid = W1.shape[2]
    out_d = W2.shape[2]

    patches = x.reshape(bn, c_in, g, p, g, p).transpose(
        0, 2, 4, 1, 3, 5).reshape(bn, n, pd)

    topk, A, agg = pl.pallas_call(
        _graph_kernel,
        grid=(bn,),
        in_specs=[
            pl.BlockSpec((1, n, pd), lambda b: (b, 0, 0)),
            pl.BlockSpec((pd, feat), lambda b: (0, 0)),
            pl.BlockSpec((1, feat), lambda b: (0, 0)),
            pl.BlockSpec((feat, c), lambda b: (0, 0)),
            pl.BlockSpec((1, c), lambda b: (0, 0)),
        ],
        out_specs=[
            pl.BlockSpec((1, 1, _TOP_K), lambda b: (b, 0, 0)),
            pl.BlockSpec((1, n, n), lambda b: (b, 0, 0)),
            pl.BlockSpec((1, n, feat), lambda b: (b, 0, 0)),
        ],
        out_shape=[
            jax.ShapeDtypeStruct((bn, 1, _TOP_K), jnp.int32),
            jax.ShapeDtypeStruct((bn, n, n), jnp.float32),
            jax.ShapeDtypeStruct((bn, n, feat), jnp.float32),
        ],
        compiler_params=pltpu.CompilerParams(
            dimension_semantics=("arbitrary",)),
    )(patches, W_patch, b_patch.reshape(1, feat), W_cls, b_cls.reshape(1, c))

    idx = topk.reshape(bn * _TOP_K)

    out = pl.pallas_call(
        _expert_kernel,
        grid_spec=pltpu.PrefetchScalarGridSpec(
            num_scalar_prefetch=1,
            grid=(bn * _TOP_K,),
            in_specs=[
                pl.BlockSpec((1, n, n), lambda i, s: (i // _TOP_K, 0, 0)),
                pl.BlockSpec((1, n, feat), lambda i, s: (i // _TOP_K, 0, 0)),
                pl.BlockSpec((1, feat, hid), lambda i, s: (s[i], 0, 0)),
                pl.BlockSpec((1, 1, hid), lambda i, s: (s[i], 0, 0)),
                pl.BlockSpec((1, hid, out_d), lambda i, s: (s[i], 0, 0)),
                pl.BlockSpec((1, 1, out_d), lambda i, s: (s[i], 0, 0)),
                pl.BlockSpec((out_d, c), lambda i, s: (0, 0)),
                pl.BlockSpec((1, c), lambda i, s: (0, 0)),
            ],
            out_specs=pl.BlockSpec((1, 1, c), lambda i, s: (i // _TOP_K, 0, 0)),
        ),
        out_shape=jax.ShapeDtypeStruct((bn, 1, c), jnp.float32),
        compiler_params=pltpu.CompilerParams(
            dimension_semantics=("arbitrary",)),
    )(idx, A, agg, W1, b1.reshape(c, 1, hid), W2, b2.reshape(c, 1, out_d),
      W_fc, b_fc.reshape(1, c))

    return out.reshape(bn, c)


# speedup vs baseline: 26.2113x; 1.7384x over previous
import jax
import jax.numpy as jnp
from jax import lax
from jax.experimental import pallas as pl
from jax.experimental.pallas import tpu as pltpu

_P = 16
_TOP_K = 2
_KNN = 8


def _graph_kernel(x_ref, Wp_ref, bp_ref, Wcls_ref, bcls_ref,
                  topk_ref, A_ref, agg_ref):
    c_in, g, p, _ = x_ref.shape[1], x_ref.shape[2], x_ref.shape[3], 0
    n = g * g
    pd = c_in * p * p
    xr = x_ref[0].reshape(c_in, g, p, g, p)
    patches = jnp.transpose(xr, (1, 3, 0, 2, 4)).reshape(n, pd)
    f = jnp.dot(patches, Wp_ref[...],
                preferred_element_type=jnp.float32) + bp_ref[...]
    n, feat = f.shape

    fm = jnp.mean(f, axis=0, keepdims=True)
    lg = jnp.dot(fm, Wcls_ref[...],
                 preferred_element_type=jnp.float32) + bcls_ref[...]
    c = lg.shape[1]
    iota_c = lax.broadcasted_iota(jnp.int32, (1, c), 1)
    m1 = jnp.max(lg, axis=1, keepdims=True)
    i1 = jnp.min(jnp.where(lg == m1, iota_c, c), axis=1, keepdims=True)
    lg2 = jnp.where(iota_c == i1, -jnp.inf, lg)
    m2 = jnp.max(lg2, axis=1, keepdims=True)
    i2 = jnp.min(jnp.where(lg2 == m2, iota_c, c), axis=1, keepdims=True)
    iota_2 = lax.broadcasted_iota(jnp.int32, (1, 1, _TOP_K), 2)
    topk_ref[...] = jnp.where(iota_2 == 0, i1.reshape(1, 1, 1),
                              i2.reshape(1, 1, 1))

    gram = lax.dot_general(f, f, (((1,), (1,)), ((), ())),
                           preferred_element_type=jnp.float32)
    rows = lax.broadcasted_iota(jnp.int32, (n, n), 0)
    cols = lax.broadcasted_iota(jnp.int32, (n, n), 1)
    eye = rows == cols
    sq_col = jnp.sum(jnp.where(eye, gram, 0.0), axis=0, keepdims=True)
    d = sq_col - 2.0 * gram
    d = jnp.where(eye, d + 1e9, d)

    work = d
    vals = []
    nbrs = []
    for _ in range(_KNN):
        m = jnp.min(work, axis=1, keepdims=True)
        am = jnp.min(jnp.where(work == m, cols, n), axis=1, keepdims=True)
        vals.append(m)
        nbrs.append(am)
        work = jnp.where(cols == am, jnp.inf, work)
    v = jnp.concatenate(vals, axis=1)

    z = -v * (1.0 / jnp.sqrt(jnp.float32(feat)))
    z = z - jnp.max(z, axis=1, keepdims=True)
    e = jnp.exp(z)
    w = e / jnp.sum(e, axis=1, keepdims=True)

    a = jnp.zeros((n, n), jnp.float32)
    for k in range(_KNN):
        a = a + jnp.where(cols == nbrs[k], w[:, k:k + 1], 0.0)
    A_ref[0] = a
    agg_ref[0] = jnp.dot(a, f, preferred_element_type=jnp.float32)


def _expert_kernel(idx_ref, A_ref, agg_ref, W1_ref, b1_ref, W2_ref, b2_ref,
                   Wfc_ref, bfc_ref, out_ref):
    k = pl.program_id(0) % _TOP_K
    a = A_ref[0]
    h = jnp.maximum(
        jnp.dot(agg_ref[0], W1_ref[0],
                preferred_element_type=jnp.float32) + b1_ref[0], 0.0)
    agg2 = jnp.dot(a, h, preferred_element_type=jnp.float32)
    o = jnp.dot(agg2, W2_ref[0],
                preferred_element_type=jnp.float32) + b2_ref[0]
    om = jnp.mean(o, axis=0, keepdims=True)
    y = (1.0 / _TOP_K) * (
        jnp.dot(om, Wfc_ref[...], preferred_element_type=jnp.float32)
        + bfc_ref[...])
    y = y.reshape(1, 1, -1)

    @pl.when(k == 0)
    def _():
        out_ref[...] = y

    @pl.when(k != 0)
    def _():
        out_ref[...] = out_ref[...] + y


def kernel(x, W_patch, b_patch, W_cls, b_cls, W1, b1, W2, b2, W_fc, b_fc):
    bn, c_in, hw, _ = x.shape
    p = _P
    g = hw // p
    n = g * g
    pd = c_in * p * p
    feat = W_patch.shape[1]
    c = W_cls.shape[1]
    hid = W1.shape[2]
    out_d = W2.shape[2]

    x4 = x.reshape(bn, c_in, g, p, hw)

    topk, A, agg = pl.pallas_call(
        _graph_kernel,
        grid=(bn,),
        in_specs=[
            pl.BlockSpec((1, c_in, g, p, hw), lambda b: (b, 0, 0, 0, 0)),
            pl.BlockSpec((pd, feat), lambda b: (0, 0)),
            pl.BlockSpec((1, feat), lambda b: (0, 0)),
            pl.BlockSpec((feat, c), lambda b: (0, 0)),
            pl.BlockSpec((1, c), lambda b: (0, 0)),
        ],
        out_specs=[
            pl.BlockSpec((1, 1, _TOP_K), lambda b: (b, 0, 0)),
            pl.BlockSpec((1, n, n), lambda b: (b, 0, 0)),
            pl.BlockSpec((1, n, feat), lambda b: (b, 0, 0)),
        ],
        out_shape=[
            jax.ShapeDtypeStruct((bn, 1, _TOP_K), jnp.int32),
            jax.ShapeDtypeStruct((bn, n, n), jnp.float32),
            jax.ShapeDtypeStruct((bn, n, feat), jnp.float32),
        ],
        compiler_params=pltpu.CompilerParams(
            dimension_semantics=("arbitrary",)),
    )(x4, W_patch, b_patch.reshape(1, feat), W_cls, b_cls.reshape(1, c))

    idx = topk.reshape(bn * _TOP_K)

    out = pl.pallas_call(
        _expert_kernel,
        grid_spec=pltpu.PrefetchScalarGridSpec(
            num_scalar_prefetch=1,
            grid=(bn * _TOP_K,),
            in_specs=[
                pl.BlockSpec((1, n, n), lambda i, s: (i // _TOP_K, 0, 0)),
                pl.BlockSpec((1, n, feat), lambda i, s: (i // _TOP_K, 0, 0)),
                pl.BlockSpec((1, feat, hid), lambda i, s: (s[i], 0, 0)),
                pl.BlockSpec((1, 1, hid), lambda i, s: (s[i], 0, 0)),
                pl.BlockSpec((1, hid, out_d), lambda i, s: (s[i], 0, 0)),
                pl.BlockSpec((1, 1, out_d), lambda i, s: (s[i], 0, 0)),
                pl.BlockSpec((out_d, c), lambda i, s: (0, 0)),
                pl.BlockSpec((1, c), lambda i, s: (0, 0)),
            ],
            out_specs=pl.BlockSpec((1, 1, c), lambda i, s: (i // _TOP_K, 0, 0)),
        ),
        out_shape=jax.ShapeDtypeStruct((bn, 1, c), jnp.float32),
        compiler_params=pltpu.CompilerParams(
            dimension_semantics=("arbitrary",)),
    )(idx, A, agg, W1, b1.reshape(c, 1, hid), W2, b2.reshape(c, 1, out_d),
      W_fc, b_fc.reshape(1, c))

    return out.reshape(bn, c)


# speedup vs baseline: 28.2747x; 1.0787x over previous
import jax
import jax.numpy as jnp
from jax import lax
from jax.experimental import pallas as pl
from jax.experimental.pallas import tpu as pltpu

_P = 16
_TOP_K = 2
_KNN = 8


def _graph_kernel(x_ref, Wp_ref, bp_ref, Wcls_ref, bcls_ref,
                  topk_ref, mA_ref, agg_ref):
    c_in, g, p = x_ref.shape[1], x_ref.shape[2], x_ref.shape[3]
    n = g * g
    pd = c_in * p * p
    xr = x_ref[0].reshape(c_in, g, p, g, p)
    patches = jnp.transpose(xr, (1, 3, 0, 2, 4)).reshape(n, pd)
    f = jnp.dot(patches, Wp_ref[...],
                preferred_element_type=jnp.float32) + bp_ref[...]
    n, feat = f.shape

    mp = jnp.mean(patches, axis=0, keepdims=True)
    fm = jnp.dot(mp, Wp_ref[...], precision=lax.Precision.HIGHEST,
                 preferred_element_type=jnp.float32) + bp_ref[...]
    lg = jnp.dot(fm, Wcls_ref[...], precision=lax.Precision.HIGHEST,
                 preferred_element_type=jnp.float32) + bcls_ref[...]
    c = lg.shape[1]
    iota_c = lax.broadcasted_iota(jnp.int32, (1, c), 1)
    m1 = jnp.max(lg, axis=1, keepdims=True)
    i1 = jnp.min(jnp.where(lg == m1, iota_c, c), axis=1, keepdims=True)
    lg2 = jnp.where(iota_c == i1, -jnp.inf, lg)
    m2 = jnp.max(lg2, axis=1, keepdims=True)
    i2 = jnp.min(jnp.where(lg2 == m2, iota_c, c), axis=1, keepdims=True)
    iota_2 = lax.broadcasted_iota(jnp.int32, (1, 1, _TOP_K), 2)
    topk_ref[...] = jnp.where(iota_2 == 0, i1.reshape(1, 1, 1),
                              i2.reshape(1, 1, 1))

    gram = lax.dot_general(f, f, (((1,), (1,)), ((), ())),
                           preferred_element_type=jnp.float32)
    rows = lax.broadcasted_iota(jnp.int32, (n, n), 0)
    cols = lax.broadcasted_iota(jnp.int32, (n, n), 1)
    eye = rows == cols
    sq_col = jnp.sum(jnp.where(eye, gram, 0.0), axis=0, keepdims=True)
    d = sq_col - 2.0 * gram
    d = jnp.where(eye, d + 1e9, d)

    work = d
    vals = []
    nbrs = []
    for _ in range(_KNN):
        m = jnp.min(work, axis=1, keepdims=True)
        am = jnp.min(jnp.where(work == m, cols, n), axis=1, keepdims=True)
        vals.append(m)
        nbrs.append(am)
        work = jnp.where(cols == am, jnp.inf, work)
    v = jnp.concatenate(vals, axis=1)

    z = -v * (1.0 / jnp.sqrt(jnp.float32(feat)))
    z = z - jnp.max(z, axis=1, keepdims=True)
    e = jnp.exp(z)
    w = e / jnp.sum(e, axis=1, keepdims=True)

    a = jnp.zeros((n, n), jnp.float32)
    for k in range(_KNN):
        a = a + jnp.where(cols == nbrs[k], w[:, k:k + 1], 0.0)
    mA_ref[0] = jnp.sum(a, axis=0, keepdims=True) * (1.0 / n)
    agg_ref[0] = jnp.dot(a, f, preferred_element_type=jnp.float32)


def _expert_kernel(idx_ref, mA_ref, agg_ref, W1_ref, b1_ref, W2_ref, b2_ref,
                   Wfc_ref, bfc_ref, out_ref):
    k = pl.program_id(0) % _TOP_K
    h = jnp.maximum(
        jnp.dot(agg_ref[0], W1_ref[0],
                preferred_element_type=jnp.float32) + b1_ref[0], 0.0)
    v = jnp.dot(mA_ref[0], h, preferred_element_type=jnp.float32)
    om = jnp.dot(v, W2_ref[0],
                 preferred_element_type=jnp.float32) + b2_ref[0]
    y = (1.0 / _TOP_K) * (
        jnp.dot(om, Wfc_ref[...], preferred_element_type=jnp.float32)
        + bfc_ref[...])
    y = y.reshape(1, 1, -1)

    @pl.when(k == 0)
    def _():
        out_ref[...] = y

    @pl.when(k != 0)
    def _():
        out_ref[...] = out_ref[...] + y


def kernel(x, W_patch, b_patch, W_cls, b_cls, W1, b1, W2, b2, W_fc, b_fc):
    bn, c_in, hw, _ = x.shape
    p = _P
    g = hw // p
    n = g * g
    pd = c_in * p * p
    feat = W_patch.shape[1]
    c = W_cls.shape[1]
    hid = W1.shape[2]
    out_d = W2.shape[2]

    x4 = x.reshape(bn, c_in, g, p, hw)

    topk, mA, agg = pl.pallas_call(
        _graph_kernel,
        grid=(bn,),
        in_specs=[
            pl.BlockSpec((1, c_in, g, p, hw), lambda b: (b, 0, 0, 0, 0)),
            pl.BlockSpec((pd, feat), lambda b: (0, 0)),
            pl.BlockSpec((1, feat), lambda b: (0, 0)),
            pl.BlockSpec((feat, c), lambda b: (0, 0)),
            pl.BlockSpec((1, c), lambda b: (0, 0)),
        ],
        out_specs=[
            pl.BlockSpec((1, 1, _TOP_K), lambda b: (b, 0, 0)),
            pl.BlockSpec((1, 1, n), lambda b: (b, 0, 0)),
            pl.BlockSpec((1, n, feat), lambda b: (b, 0, 0)),
        ],
        out_shape=[
            jax.ShapeDtypeStruct((bn, 1, _TOP_K), jnp.int32),
            jax.ShapeDtypeStruct((bn, 1, n), jnp.float32),
            jax.ShapeDtypeStruct((bn, n, feat), jnp.float32),
        ],
        compiler_params=pltpu.CompilerParams(
            dimension_semantics=("arbitrary",)),
    )(x4, W_patch, b_patch.reshape(1, feat), W_cls, b_cls.reshape(1, c))
    idx = topk.reshape(bn * _TOP_K)

    out = pl.pallas_call(
        _expert_kernel,
        grid_spec=pltpu.PrefetchScalarGridSpec(
            num_scalar_prefetch=1,
            grid=(bn * _TOP_K,),
            in_specs=[
                pl.BlockSpec((1, 1, n), lambda i, s: (i // _TOP_K, 0, 0)),
                pl.BlockSpec((1, n, feat), lambda i, s: (i // _TOP_K, 0, 0)),
                pl.BlockSpec((1, feat, hid), lambda i, s: (s[i], 0, 0)),
                pl.BlockSpec((1, 1, hid), lambda i, s: (s[i], 0, 0)),
                pl.BlockSpec((1, hid, out_d), lambda i, s: (s[i], 0, 0)),
                pl.BlockSpec((1, 1, out_d), lambda i, s: (s[i], 0, 0)),
                pl.BlockSpec((out_d, c), lambda i, s: (0, 0)),
                pl.BlockSpec((1, c), lambda i, s: (0, 0)),
            ],
            out_specs=pl.BlockSpec((1, 1, c), lambda i, s: (i // _TOP_K, 0, 0)),
        ),
        out_shape=jax.ShapeDtypeStruct((bn, 1, c), jnp.float32),
        compiler_params=pltpu.CompilerParams(
            dimension_semantics=("arbitrary",)),
    )(idx, mA, agg, W1, b1.reshape(c, 1, hid), W2, b2.reshape(c, 1, out_d),
      W_fc, b_fc.reshape(1, c))

    return out.reshape(bn, c)
